# baseline (speedup 1.0000x reference)
import functools

import jax
import jax.numpy as jnp
from jax import lax
from jax.experimental import pallas as pl
from jax.experimental.pallas import tpu as pltpu
from jax.experimental.pallas import tpu_sc as plsc

H = 64
EH = 128
NRBF = 50
GAP = 0.1
NGRAPH = 512
STEPS = 6
S2S_STEPS = 6

_NC = 2
_NS = 16
_NW = _NC * _NS
_CHUNK = 128

_INTERPRET = False


@functools.lru_cache(maxsize=None)
def _sc_gather_fn(n_chunks, d):
    iters = -(-n_chunks // _NW)
    mesh = plsc.VectorSubcoreMesh(core_axis_name="c", subcore_axis_name="s")

    @functools.partial(
        pl.kernel, mesh=mesh,
        out_type=jax.ShapeDtypeStruct((n_chunks * _CHUNK, d), jnp.float32),
        scratch_types=[
            pltpu.VMEM((_CHUNK,), jnp.int32),
            pltpu.VMEM((_CHUNK, d), jnp.float32),
            pltpu.SemaphoreType.DMA,
        ],
    )
    def k(table_h, idx_h, out_h, idx_v, rows_v, sem):
        wid = lax.axis_index("s") * _NC + lax.axis_index("c")

        def body(j, carry):
            q = wid + _NW * j

            @pl.when(q < n_chunks)
            def _():
                pltpu.sync_copy(idx_h.at[q], idx_v)
                pltpu.async_copy(table_h.at[idx_v], rows_v, sem).wait()
                pltpu.sync_copy(rows_v, out_h.at[pl.ds(q * _CHUNK, _CHUNK)])

            return carry

        lax.fori_loop(0, iters, body, 0)

    return k


def _sc_gather(table, idx2d):
    return _sc_gather_fn(idx2d.shape[0], table.shape[1])(table, idx2d)


@functools.lru_cache(maxsize=None)
def _sc_segsum_fn(n_chunks, nrows, d):
    iters = -(-n_chunks // _NW)
    rows_per_sub = 1000
    n_sub_rows = nrows // rows_per_sub
    mesh = plsc.VectorSubcoreMesh(core_axis_name="c", subcore_axis_name="s")

    @functools.partial(
        pl.kernel, mesh=mesh,
        out_type=jax.ShapeDtypeStruct((_NC, nrows, d), jnp.float32),
        scratch_types=[
            pltpu.VMEM((_CHUNK,), jnp.int32),
            pltpu.VMEM((_CHUNK, d), jnp.float32),
            pltpu.VMEM_SHARED((nrows, d), jnp.float32),
        ],
    )
    def k(vals_h, idx_h, zeros_h, out_h, idx_v, rows_v, acc_s):
        c = lax.axis_index("c")
        s = lax.axis_index("s")
        wid = s * _NC + c
        @pl.when(s < n_sub_rows)
        def _():
            pltpu.sync_copy(zeros_h,
                            acc_s.at[pl.ds(s * rows_per_sub, rows_per_sub)])

        plsc.subcore_barrier()

        def body(j, carry):
            q = wid + _NW * j

            @pl.when(q < n_chunks)
            def _():
                pltpu.sync_copy(idx_h.at[q], idx_v)
                pltpu.sync_copy(vals_h.at[pl.ds(q * _CHUNK, _CHUNK)], rows_v)
                pltpu.sync_copy(rows_v, acc_s.at[idx_v], add=True)

            return carry

        lax.fori_loop(0, iters, body, 0)
        plsc.subcore_barrier()

        @pl.when(s < n_sub_rows)
        def _():
            pltpu.sync_copy(
                acc_s.at[pl.ds(s * rows_per_sub, rows_per_sub)],
                out_h.at[c, pl.ds(s * rows_per_sub, rows_per_sub)],
            )

    return k


def _sc_segsum(vals, idx2d, zeros_sub, nrows):
    return _sc_segsum_fn(idx2d.shape[0], nrows, vals.shape[1])(vals, idx2d, zeros_sub)


def _h0_body(x_ref, w_ref, b_ref, o_ref):
    o_ref[...] = jax.nn.relu(
        jnp.dot(x_ref[...], w_ref[...], preferred_element_type=jnp.float32)
        + b_ref[...])


def _tc_h0(x, w0t, b0):
    n, f = x.shape
    nb = 400
    return pl.pallas_call(
        _h0_body,
        grid=(n // nb,),
        in_specs=[
            pl.BlockSpec((nb, f), lambda i: (i, 0)),
            pl.BlockSpec((f, H), lambda i: (0, 0)),
            pl.BlockSpec((1, H), lambda i: (0, 0)),
        ],
        out_specs=pl.BlockSpec((nb, H), lambda i: (i, 0)),
        out_shape=jax.ShapeDtypeStruct((n, H), jnp.float32),
        interpret=_INTERPRET,
    )(x, w0t, b0)


def _edge_net_body(ea_ref, w1a_ref, w1b_ref, b1_ref, w2_ref, b2_ref, o_ref):
    ea = ea_ref[...]
    d = ea[:, 15:16]
    centers = lax.broadcasted_iota(jnp.int32, (1, NRBF), 1).astype(jnp.float32) * GAP
    rbf = jnp.exp(-((d - centers) ** 2) / (GAP * GAP))
    hid = jax.nn.relu(
        jnp.dot(ea[:, :15], w1a_ref[...], preferred_element_type=jnp.float32)
        + jnp.dot(rbf, w1b_ref[...], preferred_element_type=jnp.float32)
        + b1_ref[...])
    o_ref[...] = (
        jnp.dot(hid, w2_ref[...], preferred_element_type=jnp.float32)
        + b2_ref[...])


def _tc_edge_net(edge_attr, w1at, w1bt, be1, w2t, be2):
    e = edge_attr.shape[0]
    eb = 320
    return pl.pallas_call(
        _edge_net_body,
        grid=(e // eb,),
        in_specs=[
            pl.BlockSpec((eb, 16), lambda i: (i, 0)),
            pl.BlockSpec((15, EH), lambda i: (0, 0)),
            pl.BlockSpec((NRBF, EH), lambda i: (0, 0)),
            pl.BlockSpec((1, EH), lambda i: (0, 0)),
            pl.BlockSpec((EH, H * H), lambda i: (0, 0)),
            pl.BlockSpec((1, H * H), lambda i: (0, 0)),
        ],
        out_specs=pl.BlockSpec((eb, H * H), lambda i: (i, 0)),
        out_shape=jax.ShapeDtypeStruct((e, H * H), jnp.float32),
        interpret=_INTERPRET,
    )(edge_attr, w1at, w1bt, be1, w2t, be2)


def _bmm_body(xj_ref, w_ref, o_ref):
    xj = xj_ref[:, 0:H]
    w = w_ref[...]
    acc = xj[:, 0:1] * w[:, 0:H]
    for i in range(1, H):
        acc = acc + xj[:, i:i + 1] * w[:, i * H:(i + 1) * H]
    o_ref[:, 0:H] = acc
    o_ref[:, H:2 * H] = jnp.zeros_like(acc)


def _tc_bmm(xj, w):
    e = xj.shape[0]
    eb = 256
    return pl.pallas_call(
        _bmm_body,
        grid=(e // eb,),
        in_specs=[
            pl.BlockSpec((eb, 2 * H), lambda i: (i, 0)),
            pl.BlockSpec((eb, H * H), lambda i: (i, 0)),
        ],
        out_specs=pl.BlockSpec((eb, 2 * H), lambda i: (i, 0)),
        out_shape=jax.ShapeDtypeStruct((e, 2 * H), jnp.float32),
        interpret=_INTERPRET,
    )(xj, w)


def _invcnt_body(ca_ref, cb_ref, o_ref):
    o_ref[...] = 1.0 / jnp.maximum(ca_ref[...] + cb_ref[...], 1.0)


def _tc_invcnt(ca, cb):
    n = ca.shape[0]
    nb = 400
    return pl.pallas_call(
        _invcnt_body,
        grid=(n // nb,),
        in_specs=[
            pl.BlockSpec((nb, H), lambda i: (i, 0)),
            pl.BlockSpec((nb, H), lambda i: (i, 0)),
        ],
        out_specs=pl.BlockSpec((nb, H), lambda i: (i, 0)),
        out_shape=jax.ShapeDtypeStruct((n, H), jnp.float32),
        interpret=_INTERPRET,
    )(ca, cb)


def _gru_body(aa_ref, ab_ref, inv_ref, h_ref, bc_ref, wih_ref, whh_ref,
              bih_ref, bhh_ref, o_ref):
    m = jax.nn.relu((aa_ref[...] + ab_ref[...]) * inv_ref[...] + bc_ref[...])
    h = h_ref[...]
    gi = jnp.dot(m, wih_ref[...], preferred_element_type=jnp.float32) + bih_ref[...]
    gh = jnp.dot(h, whh_ref[...], preferred_element_type=jnp.float32) + bhh_ref[...]
    r = jax.nn.sigmoid(gi[:, 0:H] + gh[:, 0:H])
    z = jax.nn.sigmoid(gi[:, H:2 * H] + gh[:, H:2 * H])
    nn = jnp.tanh(gi[:, 2 * H:3 * H] + r * gh[:, 2 * H:3 * H])
    o_ref[...] = (1.0 - z) * nn + z * h


def _tc_gru(aa, ab, inv, h, bc, wiht, whht, bih, bhh):
    n = h.shape[0]
    nb = 400
    return pl.pallas_call(
        _gru_body,
        grid=(n // nb,),
        in_specs=[
            pl.BlockSpec((nb, H), lambda i: (i, 0)),
            pl.BlockSpec((nb, H), lambda i: (i, 0)),
            pl.BlockSpec((nb, H), lambda i: (i, 0)),
            pl.BlockSpec((nb, H), lambda i: (i, 0)),
            pl.BlockSpec((1, H), lambda i: (0, 0)),
            pl.BlockSpec((H, 3 * H), lambda i: (0, 0)),
            pl.BlockSpec((H, 3 * H), lambda i: (0, 0)),
            pl.BlockSpec((1, 3 * H), lambda i: (0, 0)),
            pl.BlockSpec((1, 3 * H), lambda i: (0, 0)),
        ],
        out_specs=pl.BlockSpec((nb, H), lambda i: (i, 0)),
        out_shape=jax.ShapeDtypeStruct((n, H), jnp.float32),
        interpret=_INTERPRET,
    )(aa, ab, inv, h, bc, wiht, whht, bih, bhh)


def _s2s_body(out_ref, batch_ref, wia_ref, wib_ref, whh_ref, bi_ref, bh_ref,
              w1a_ref, w1b_ref, b1_ref, w2_ref, b2_ref, y_ref):
    n = out_ref.shape[0]
    nb = 2000
    nblk = n // nb
    gio = lax.broadcasted_iota(jnp.int32, (nb, NGRAPH), 1)

    def s2s_step(_, carry):
        q, r_read, hs, cs = carry
        gates = (jnp.dot(q, wia_ref[...], preferred_element_type=jnp.float32)
                 + jnp.dot(r_read, wib_ref[...], preferred_element_type=jnp.float32)
                 + jnp.dot(hs, whh_ref[...], preferred_element_type=jnp.float32)
                 + bi_ref[...] + bh_ref[...])
        ig = jax.nn.sigmoid(gates[:, 0:H])
        fg = jax.nn.sigmoid(gates[:, H:2 * H])
        gg = jnp.tanh(gates[:, 2 * H:3 * H])
        og = jax.nn.sigmoid(gates[:, 3 * H:4 * H])
        cs = fg * cs + ig * gg
        hs = og * jnp.tanh(cs)
        q = hs

        def p1(b, emax):
            sl = pl.ds(b * nb, nb)
            mask = (batch_ref[sl, :] == gio).astype(jnp.float32)
            qb = jnp.dot(mask, q, preferred_element_type=jnp.float32)
            e_b = jnp.sum(out_ref[sl, :] * qb, axis=1, keepdims=True)
            masked = jnp.where(mask > 0.0, e_b, -1e30)
            return jnp.maximum(emax, jnp.max(masked, axis=0, keepdims=True))

        emax = lax.fori_loop(0, nblk, p1,
                             jnp.full((1, NGRAPH), -1e30, jnp.float32))

        def p2(b, c2):
            denom, num = c2
            sl = pl.ds(b * nb, nb)
            mask = (batch_ref[sl, :] == gio).astype(jnp.float32)
            qb = jnp.dot(mask, q, preferred_element_type=jnp.float32)
            e_b = jnp.sum(out_ref[sl, :] * qb, axis=1, keepdims=True)
            emax_b = jnp.sum(mask * emax, axis=1, keepdims=True)
            ee_b = jnp.exp(e_b - emax_b)
            denom = denom + lax.dot_general(
                mask, ee_b, (((0,), (0,)), ((), ())),
                preferred_element_type=jnp.float32)
            num = num + lax.dot_general(
                mask * ee_b, out_ref[sl, :], (((0,), (0,)), ((), ())),
                preferred_element_type=jnp.float32)
            return denom, num

        denom, num = lax.fori_loop(
            0, nblk, p2, (jnp.zeros((NGRAPH, 1), jnp.float32),
                          jnp.zeros((NGRAPH, H), jnp.float32)))
        r_read = num / jnp.maximum(denom, 1e-30)
        return q, r_read, hs, cs

    zg = jnp.zeros((NGRAPH, H), jnp.float32)
    q, r_read, hs, cs = lax.fori_loop(0, S2S_STEPS, s2s_step, (zg, zg, zg, zg))
    y1 = jax.nn.relu(
        jnp.dot(q, w1a_ref[...], preferred_element_type=jnp.float32)
        + jnp.dot(r_read, w1b_ref[...], preferred_element_type=jnp.float32)
        + b1_ref[...])
    y_ref[...] = (
        jnp.dot(y1, w2_ref[...], preferred_element_type=jnp.float32)
        + b2_ref[...])


def _tc_s2s(out, batch_col, wiat, wibt, whht, bih, bhh, w1at, w1bt, b1, w2t, b2):
    n = out.shape[0]
    odim = w2t.shape[1]
    return pl.pallas_call(
        _s2s_body,
        in_specs=[
            pl.BlockSpec((n, H), lambda: (0, 0)),
            pl.BlockSpec((n, 1), lambda: (0, 0)),
            pl.BlockSpec((H, 4 * H), lambda: (0, 0)),
            pl.BlockSpec((H, 4 * H), lambda: (0, 0)),
            pl.BlockSpec((H, 4 * H), lambda: (0, 0)),
            pl.BlockSpec((1, 4 * H), lambda: (0, 0)),
            pl.BlockSpec((1, 4 * H), lambda: (0, 0)),
            pl.BlockSpec((H, H), lambda: (0, 0)),
            pl.BlockSpec((H, H), lambda: (0, 0)),
            pl.BlockSpec((1, H), lambda: (0, 0)),
            pl.BlockSpec((H, odim), lambda: (0, 0)),
            pl.BlockSpec((1, odim), lambda: (0, 0)),
        ],
        out_specs=pl.BlockSpec((NGRAPH, odim), lambda: (0, 0)),
        out_shape=jax.ShapeDtypeStruct((NGRAPH, odim), jnp.float32),
        interpret=_INTERPRET,
    )(out, batch_col, wiat, wibt, whht, bih, bhh, w1at, w1bt, b1, w2t, b2)


def kernel(x, edge_attr, edge_index, batch, W0, b0, We1, be1, We2, be2,
           b_conv, gru_Wih, gru_Whh, gru_bih, gru_bhh, lstm_Wih, lstm_Whh,
           lstm_bih, lstm_bhh, W1, b1, W2, b2):
    n = x.shape[0]
    e = edge_attr.shape[0]

    src2 = edge_index[0].reshape(-1, _CHUNK)
    dst2 = edge_index[1].reshape(-1, _CHUNK)
    batch_col = batch[:, None]
    zeros_sub = jnp.zeros((1000, 2 * H), jnp.float32)
    ones_e = jnp.ones((e, 2 * H), jnp.float32)
    pad_n = jnp.zeros((n, H), jnp.float32)

    w0t = W0.T
    w1at = We1[:, :15].T
    w1bt = We1[:, 15:].T
    we2t = We2.T
    wiht = gru_Wih.T
    whht = gru_Whh.T
    lwiat = lstm_Wih[:, :H].T
    lwibt = lstm_Wih[:, H:].T
    lwhht = lstm_Whh.T
    w1at_f = W1[:, :H].T
    w1bt_f = W1[:, H:].T
    w2t = W2.T

    out = _tc_h0(x, w0t, b0[None, :])
    w = _tc_edge_net(edge_attr, w1at, w1bt, be1[None, :], we2t, be2[None, :])
    cnt2 = _sc_segsum(ones_e, dst2, zeros_sub, n)
    inv = _tc_invcnt(cnt2[0, :, :H], cnt2[1, :, :H])

    def mpnn_step(_, out):
        table = jnp.concatenate([out, pad_n], axis=1)
        xj = _sc_gather(table, src2)
        msg = _tc_bmm(xj, w)
        ag2 = _sc_segsum(msg, dst2, zeros_sub, n)
        return _tc_gru(ag2[0, :, :H], ag2[1, :, :H], inv, out, b_conv[None, :],
                       wiht, whht, gru_bih[None, :], gru_bhh[None, :])

    out = lax.fori_loop(0, STEPS, mpnn_step, out)

    y = _tc_s2s(out, batch_col, lwiat, lwibt, lwhht, lstm_bih[None, :],
                lstm_bhh[None, :], w1at_f, w1bt_f, b1[None, :], w2t,
                b2[None, :])
    return y


# speedup vs baseline: 2.2905x; 2.2905x over previous
import functools

import jax
import jax.numpy as jnp
from jax import lax
from jax.experimental import pallas as pl
from jax.experimental.pallas import tpu as pltpu
from jax.experimental.pallas import tpu_sc as plsc

H = 64
EH = 128
NRBF = 50
GAP = 0.1
NGRAPH = 512
STEPS = 6
S2S_STEPS = 6

_NC = 2
_NS = 16
_NW = _NC * _NS
_CHUNK = 128

_INTERPRET = False


@functools.lru_cache(maxsize=None)
def _sc_gather_fn(n_chunks, d):
    iters = -(-n_chunks // _NW)
    mesh = plsc.VectorSubcoreMesh(core_axis_name="c", subcore_axis_name="s")

    @functools.partial(
        pl.kernel, mesh=mesh,
        out_type=jax.ShapeDtypeStruct((n_chunks * _CHUNK, d), jnp.float32),
        scratch_types=[
            pltpu.VMEM((_CHUNK,), jnp.int32),
            pltpu.VMEM((_CHUNK, d), jnp.float32),
            pltpu.SemaphoreType.DMA,
        ],
    )
    def k(table_h, idx_h, out_h, idx_v, rows_v, sem):
        wid = lax.axis_index("s") * _NC + lax.axis_index("c")

        def body(j, carry):
            q = wid + _NW * j

            @pl.when(q < n_chunks)
            def _():
                pltpu.sync_copy(idx_h.at[q], idx_v)
                pltpu.async_copy(table_h.at[idx_v], rows_v, sem).wait()
                pltpu.sync_copy(rows_v, out_h.at[pl.ds(q * _CHUNK, _CHUNK)])

            return carry

        lax.fori_loop(0, iters, body, 0)

    return k


def _sc_gather(table, idx2d):
    return _sc_gather_fn(idx2d.shape[0], table.shape[1])(table, idx2d)


@functools.lru_cache(maxsize=None)
def _sc_segsum_fn(n_chunks, nrows, d):
    iters = -(-n_chunks // _NW)
    rows_per_sub = 1000
    n_sub_rows = nrows // rows_per_sub
    mesh = plsc.VectorSubcoreMesh(core_axis_name="c", subcore_axis_name="s")

    @functools.partial(
        pl.kernel, mesh=mesh,
        out_type=jax.ShapeDtypeStruct((_NC, nrows, d), jnp.float32),
        scratch_types=[
            pltpu.VMEM((_CHUNK,), jnp.int32),
            pltpu.VMEM((_CHUNK, d), jnp.float32),
            pltpu.VMEM_SHARED((nrows, d), jnp.float32),
        ],
    )
    def k(vals_h, idx_h, zeros_h, out_h, idx_v, rows_v, acc_s):
        c = lax.axis_index("c")
        s = lax.axis_index("s")
        wid = s * _NC + c
        @pl.when(s < n_sub_rows)
        def _():
            pltpu.sync_copy(zeros_h,
                            acc_s.at[pl.ds(s * rows_per_sub, rows_per_sub)])

        plsc.subcore_barrier()

        def body(j, carry):
            q = wid + _NW * j

            @pl.when(q < n_chunks)
            def _():
                pltpu.sync_copy(idx_h.at[q], idx_v)
                pltpu.sync_copy(vals_h.at[pl.ds(q * _CHUNK, _CHUNK)], rows_v)
                pltpu.sync_copy(rows_v, acc_s.at[idx_v], add=True)

            return carry

        lax.fori_loop(0, iters, body, 0)
        plsc.subcore_barrier()

        @pl.when(s < n_sub_rows)
        def _():
            pltpu.sync_copy(
                acc_s.at[pl.ds(s * rows_per_sub, rows_per_sub)],
                out_h.at[c, pl.ds(s * rows_per_sub, rows_per_sub)],
            )

    return k


def _sc_segsum(vals, idx2d, zeros_sub, nrows):
    return _sc_segsum_fn(idx2d.shape[0], nrows, vals.shape[1])(vals, idx2d, zeros_sub)


def _h0_body(x_ref, w_ref, b_ref, o_ref):
    o_ref[...] = jax.nn.relu(
        jnp.dot(x_ref[...], w_ref[...], preferred_element_type=jnp.float32)
        + b_ref[...])


def _tc_h0(x, w0t, b0):
    n, f = x.shape
    nb = 400
    return pl.pallas_call(
        _h0_body,
        grid=(n // nb,),
        in_specs=[
            pl.BlockSpec((nb, f), lambda i: (i, 0)),
            pl.BlockSpec((f, H), lambda i: (0, 0)),
            pl.BlockSpec((1, H), lambda i: (0, 0)),
        ],
        out_specs=pl.BlockSpec((nb, H), lambda i: (i, 0)),
        out_shape=jax.ShapeDtypeStruct((n, H), jnp.float32),
        interpret=_INTERPRET,
    )(x, w0t, b0)


def _edge_net_body(ea_ref, w1a_ref, w1b_ref, b1_ref, w2_ref, b2_ref, o_ref):
    ea = ea_ref[...]
    d = ea[:, 15:16]
    centers = lax.broadcasted_iota(jnp.int32, (1, NRBF), 1).astype(jnp.float32) * GAP
    rbf = jnp.exp(-((d - centers) ** 2) / (GAP * GAP))
    hid = jax.nn.relu(
        jnp.dot(ea[:, :15], w1a_ref[...], preferred_element_type=jnp.float32)
        + jnp.dot(rbf, w1b_ref[...], preferred_element_type=jnp.float32)
        + b1_ref[...])
    o_ref[...] = (
        jnp.dot(hid, w2_ref[...], preferred_element_type=jnp.float32)
        + b2_ref[...])


def _tc_edge_net(edge_attr, w1at, w1bt, be1, w2t, be2):
    e = edge_attr.shape[0]
    eb = 320
    return pl.pallas_call(
        _edge_net_body,
        grid=(e // eb,),
        in_specs=[
            pl.BlockSpec((eb, 16), lambda i: (i, 0)),
            pl.BlockSpec((15, EH), lambda i: (0, 0)),
            pl.BlockSpec((NRBF, EH), lambda i: (0, 0)),
            pl.BlockSpec((1, EH), lambda i: (0, 0)),
            pl.BlockSpec((EH, H * H), lambda i: (0, 0)),
            pl.BlockSpec((1, H * H), lambda i: (0, 0)),
        ],
        out_specs=pl.BlockSpec((eb, H * H), lambda i: (i, 0)),
        out_shape=jax.ShapeDtypeStruct((e, H * H), jnp.float32),
        interpret=_INTERPRET,
    )(edge_attr, w1at, w1bt, be1, w2t, be2)


def _bmm_body(xj_ref, r_ref, w_ref, o_ref):
    w = w_ref[...]
    m = jnp.dot(xj_ref[...], r_ref[...], preferred_element_type=jnp.float32)
    acc = m[:, 0:2 * H] * w[:, 0:2 * H]
    for i in range(1, H // 2):
        sl = slice(i * 2 * H, (i + 1) * 2 * H)
        acc = acc + m[:, sl] * w[:, sl]
    o_ref[:, 0:H] = acc[:, 0:H] + acc[:, H:2 * H]
    o_ref[:, H:2 * H] = jnp.zeros((acc.shape[0], H), jnp.float32)


def _pair_expand_mat():
    c = jnp.arange(H * H)
    k = 2 * (c // (2 * H)) + (c % (2 * H) >= H)
    return (jnp.arange(2 * H)[:, None] == k[None, :]).astype(jnp.float32)


def _tc_bmm(xj, rmat, w):
    e = xj.shape[0]
    eb = 256
    return pl.pallas_call(
        _bmm_body,
        grid=(e // eb,),
        in_specs=[
            pl.BlockSpec((eb, 2 * H), lambda i: (i, 0)),
            pl.BlockSpec((2 * H, H * H), lambda i: (0, 0)),
            pl.BlockSpec((eb, H * H), lambda i: (i, 0)),
        ],
        out_specs=pl.BlockSpec((eb, 2 * H), lambda i: (i, 0)),
        out_shape=jax.ShapeDtypeStruct((e, 2 * H), jnp.float32),
        interpret=_INTERPRET,
    )(xj, rmat, w)


def _invcnt_body(ca_ref, cb_ref, o_ref):
    o_ref[...] = 1.0 / jnp.maximum(ca_ref[...] + cb_ref[...], 1.0)


def _tc_invcnt(ca, cb):
    n = ca.shape[0]
    nb = 400
    return pl.pallas_call(
        _invcnt_body,
        grid=(n // nb,),
        in_specs=[
            pl.BlockSpec((nb, H), lambda i: (i, 0)),
            pl.BlockSpec((nb, H), lambda i: (i, 0)),
        ],
        out_specs=pl.BlockSpec((nb, H), lambda i: (i, 0)),
        out_shape=jax.ShapeDtypeStruct((n, H), jnp.float32),
        interpret=_INTERPRET,
    )(ca, cb)


def _gru_body(aa_ref, ab_ref, inv_ref, h_ref, bc_ref, wih_ref, whh_ref,
              bih_ref, bhh_ref, o_ref):
    m = jax.nn.relu((aa_ref[...] + ab_ref[...]) * inv_ref[...] + bc_ref[...])
    h = h_ref[...]
    gi = jnp.dot(m, wih_ref[...], preferred_element_type=jnp.float32) + bih_ref[...]
    gh = jnp.dot(h, whh_ref[...], preferred_element_type=jnp.float32) + bhh_ref[...]
    r = jax.nn.sigmoid(gi[:, 0:H] + gh[:, 0:H])
    z = jax.nn.sigmoid(gi[:, H:2 * H] + gh[:, H:2 * H])
    nn = jnp.tanh(gi[:, 2 * H:3 * H] + r * gh[:, 2 * H:3 * H])
    o_ref[...] = (1.0 - z) * nn + z * h


def _tc_gru(aa, ab, inv, h, bc, wiht, whht, bih, bhh):
    n = h.shape[0]
    nb = 400
    return pl.pallas_call(
        _gru_body,
        grid=(n // nb,),
        in_specs=[
            pl.BlockSpec((nb, H), lambda i: (i, 0)),
            pl.BlockSpec((nb, H), lambda i: (i, 0)),
            pl.BlockSpec((nb, H), lambda i: (i, 0)),
            pl.BlockSpec((nb, H), lambda i: (i, 0)),
            pl.BlockSpec((1, H), lambda i: (0, 0)),
            pl.BlockSpec((H, 3 * H), lambda i: (0, 0)),
            pl.BlockSpec((H, 3 * H), lambda i: (0, 0)),
            pl.BlockSpec((1, 3 * H), lambda i: (0, 0)),
            pl.BlockSpec((1, 3 * H), lambda i: (0, 0)),
        ],
        out_specs=pl.BlockSpec((nb, H), lambda i: (i, 0)),
        out_shape=jax.ShapeDtypeStruct((n, H), jnp.float32),
        interpret=_INTERPRET,
    )(aa, ab, inv, h, bc, wiht, whht, bih, bhh)


def _s2s_body(out_ref, batch_ref, wia_ref, wib_ref, whh_ref, bi_ref, bh_ref,
              w1a_ref, w1b_ref, b1_ref, w2_ref, b2_ref, y_ref):
    n = out_ref.shape[0]
    nb = 2000
    nblk = n // nb
    gio = lax.broadcasted_iota(jnp.int32, (nb, NGRAPH), 1)

    def s2s_step(_, carry):
        q, r_read, hs, cs = carry
        gates = (jnp.dot(q, wia_ref[...], preferred_element_type=jnp.float32)
                 + jnp.dot(r_read, wib_ref[...], preferred_element_type=jnp.float32)
                 + jnp.dot(hs, whh_ref[...], preferred_element_type=jnp.float32)
                 + bi_ref[...] + bh_ref[...])
        ig = jax.nn.sigmoid(gates[:, 0:H])
        fg = jax.nn.sigmoid(gates[:, H:2 * H])
        gg = jnp.tanh(gates[:, 2 * H:3 * H])
        og = jax.nn.sigmoid(gates[:, 3 * H:4 * H])
        cs = fg * cs + ig * gg
        hs = og * jnp.tanh(cs)
        q = hs

        def p1(b, emax):
            sl = pl.ds(b * nb, nb)
            mask = (batch_ref[sl, :] == gio).astype(jnp.float32)
            qb = jnp.dot(mask, q, preferred_element_type=jnp.float32)
            e_b = jnp.sum(out_ref[sl, :] * qb, axis=1, keepdims=True)
            masked = jnp.where(mask > 0.0, e_b, -1e30)
            return jnp.maximum(emax, jnp.max(masked, axis=0, keepdims=True))

        emax = lax.fori_loop(0, nblk, p1,
                             jnp.full((1, NGRAPH), -1e30, jnp.float32))

        def p2(b, c2):
            denom, num = c2
            sl = pl.ds(b * nb, nb)
            mask = (batch_ref[sl, :] == gio).astype(jnp.float32)
            qb = jnp.dot(mask, q, preferred_element_type=jnp.float32)
            e_b = jnp.sum(out_ref[sl, :] * qb, axis=1, keepdims=True)
            emax_b = jnp.sum(mask * emax, axis=1, keepdims=True)
            ee_b = jnp.exp(e_b - emax_b)
            denom = denom + lax.dot_general(
                mask, ee_b, (((0,), (0,)), ((), ())),
                preferred_element_type=jnp.float32)
            num = num + lax.dot_general(
                mask * ee_b, out_ref[sl, :], (((0,), (0,)), ((), ())),
                preferred_element_type=jnp.float32)
            return denom, num

        denom, num = lax.fori_loop(
            0, nblk, p2, (jnp.zeros((NGRAPH, 1), jnp.float32),
                          jnp.zeros((NGRAPH, H), jnp.float32)))
        r_read = num / jnp.maximum(denom, 1e-30)
        return q, r_read, hs, cs

    zg = jnp.zeros((NGRAPH, H), jnp.float32)
    q, r_read, hs, cs = lax.fori_loop(0, S2S_STEPS, s2s_step, (zg, zg, zg, zg))
    y1 = jax.nn.relu(
        jnp.dot(q, w1a_ref[...], preferred_element_type=jnp.float32)
        + jnp.dot(r_read, w1b_ref[...], preferred_element_type=jnp.float32)
        + b1_ref[...])
    y_ref[...] = (
        jnp.dot(y1, w2_ref[...], preferred_element_type=jnp.float32)
        + b2_ref[...])


def _tc_s2s(out, batch_col, wiat, wibt, whht, bih, bhh, w1at, w1bt, b1, w2t, b2):
    n = out.shape[0]
    odim = w2t.shape[1]
    return pl.pallas_call(
        _s2s_body,
        in_specs=[
            pl.BlockSpec((n, H), lambda: (0, 0)),
            pl.BlockSpec((n, 1), lambda: (0, 0)),
            pl.BlockSpec((H, 4 * H), lambda: (0, 0)),
            pl.BlockSpec((H, 4 * H), lambda: (0, 0)),
            pl.BlockSpec((H, 4 * H), lambda: (0, 0)),
            pl.BlockSpec((1, 4 * H), lambda: (0, 0)),
            pl.BlockSpec((1, 4 * H), lambda: (0, 0)),
            pl.BlockSpec((H, H), lambda: (0, 0)),
            pl.BlockSpec((H, H), lambda: (0, 0)),
            pl.BlockSpec((1, H), lambda: (0, 0)),
            pl.BlockSpec((H, odim), lambda: (0, 0)),
            pl.BlockSpec((1, odim), lambda: (0, 0)),
        ],
        out_specs=pl.BlockSpec((NGRAPH, odim), lambda: (0, 0)),
        out_shape=jax.ShapeDtypeStruct((NGRAPH, odim), jnp.float32),
        interpret=_INTERPRET,
    )(out, batch_col, wiat, wibt, whht, bih, bhh, w1at, w1bt, b1, w2t, b2)


def kernel(x, edge_attr, edge_index, batch, W0, b0, We1, be1, We2, be2,
           b_conv, gru_Wih, gru_Whh, gru_bih, gru_bhh, lstm_Wih, lstm_Whh,
           lstm_bih, lstm_bhh, W1, b1, W2, b2):
    n = x.shape[0]
    e = edge_attr.shape[0]

    src2 = edge_index[0].reshape(-1, _CHUNK)
    dst2 = edge_index[1].reshape(-1, _CHUNK)
    batch_col = batch[:, None]
    zeros_sub = jnp.zeros((1000, 2 * H), jnp.float32)
    ones_e = jnp.ones((e, 2 * H), jnp.float32)
    pad_n = jnp.zeros((n, H), jnp.float32)

    w0t = W0.T
    w1at = We1[:, :15].T
    w1bt = We1[:, 15:].T
    we2t = We2.T
    wiht = gru_Wih.T
    whht = gru_Whh.T
    lwiat = lstm_Wih[:, :H].T
    lwibt = lstm_Wih[:, H:].T
    lwhht = lstm_Whh.T
    w1at_f = W1[:, :H].T
    w1bt_f = W1[:, H:].T
    w2t = W2.T

    rmat = _pair_expand_mat()
    out = _tc_h0(x, w0t, b0[None, :])
    w = _tc_edge_net(edge_attr, w1at, w1bt, be1[None, :], we2t, be2[None, :])
    cnt2 = _sc_segsum(ones_e, dst2, zeros_sub, n)
    inv = _tc_invcnt(cnt2[0, :, :H], cnt2[1, :, :H])

    def mpnn_step(_, out):
        table = jnp.concatenate([out, pad_n], axis=1)
        xj = _sc_gather(table, src2)
        msg = _tc_bmm(xj, rmat, w)
        ag2 = _sc_segsum(msg, dst2, zeros_sub, n)
        return _tc_gru(ag2[0, :, :H], ag2[1, :, :H], inv, out, b_conv[None, :],
                       wiht, whht, gru_bih[None, :], gru_bhh[None, :])

    out = lax.fori_loop(0, STEPS, mpnn_step, out)

    y = _tc_s2s(out, batch_col, lwiat, lwibt, lwhht, lstm_bih[None, :],
                lstm_bhh[None, :], w1at_f, w1bt_f, b1[None, :], w2t,
                b2[None, :])
    return y
